# baseline (speedup 1.0000x reference)
import functools

import numpy as np

import jax
import jax.numpy as jnp
from jax import lax
from jax.experimental import pallas as pl
from jax.experimental.pallas import tpu as pltpu
from jax.experimental.pallas import tpu_sc as plsc

NUM_EMB = 1000000
DIM = 64
PDIM = 2 * DIM
B, F = 16384, 26
FLAT = B * F
NC, NS = 2, 16
NW = NC * NS

SW = B // NW
SCH = 4
CHUNK = SCH * F
NCHB = SW // SCH
NBUF = 2

GROUP_OFFS = tuple(range(0, CHUNK - 16, 16)) + (CHUNK - 16,)

_mesh = plsc.VectorSubcoreMesh(core_axis_name="c", subcore_axis_name="s")


@functools.partial(
    pl.kernel,
    out_type=jax.ShapeDtypeStruct((B, F, DIM), jnp.float32),
    mesh=_mesh,
    scratch_types=[
        pltpu.VMEM((NCHB, CHUNK), jnp.int32),
        pltpu.VMEM((NBUF, CHUNK), jnp.int32),
        pltpu.VMEM((NBUF, CHUNK, PDIM), jnp.float32),
        pltpu.VMEM((NBUF, CHUNK, DIM), jnp.float32),
        pltpu.SemaphoreType.DMA((NBUF,)),
        pltpu.SemaphoreType.DMA((NBUF,)),
    ],
    compiler_params=pltpu.CompilerParams(use_tc_tiling_on_sc=True,
                                         needs_layout_passes=False),
)
def _gather_kernel(idx_hbm, tabd_hbm, out_hbm, idx_v, pair_v, rows_v,
                   samp_v, gsem, ssem):
    wid = lax.axis_index("s") * NC + lax.axis_index("c")
    s_base = wid * SW

    pltpu.sync_copy(idx_hbm.at[wid], idx_v)

    def make_pairs(chunk, buf):
        for o in GROUP_OFFS:
            v = idx_v[chunk, pl.ds(o, 16)]
            pair_v[buf, pl.ds(o, 16)] = v >> 1

    def gather_start(buf):
        pltpu.async_copy(tabd_hbm.at[pair_v.at[buf]], rows_v.at[buf],
                         gsem.at[buf])

    def gather_wait(buf):
        pltpu.make_async_copy(tabd_hbm.at[pair_v.at[buf]], rows_v.at[buf],
                              gsem.at[buf]).wait()

    def store_start(chunk, buf):
        pltpu.async_copy(samp_v.at[buf].reshape(SCH, F, DIM),
                         out_hbm.at[pl.ds(s_base + chunk * SCH, SCH)],
                         ssem.at[buf])

    def store_wait(chunk, buf):
        pltpu.make_async_copy(samp_v.at[buf].reshape(SCH, F, DIM),
                              out_hbm.at[pl.ds(s_base + chunk * SCH, SCH)],
                              ssem.at[buf]).wait()

    def extract(chunk, buf):
        rows2d = rows_v.at[buf]
        samp2d = samp_v.at[buf]
        for o in GROUP_OFFS:
            row_ids = jax.lax.iota(jnp.int32, 16) + o
            zero = row_ids * 0
            odd = idx_v[chunk, pl.ds(o, 16)] & 1
            col0 = odd * DIM
            for d in range(DIM):
                vals = plsc.load_gather(rows2d, [row_ids, col0 + d])
                plsc.store_scatter(samp2d, [row_ids, zero + d], vals)

    for b in range(NBUF):
        make_pairs(b, b)
        gather_start(b)

    @pl.loop(0, NCHB)
    def _chunk(c):
        b = c % NBUF
        gather_wait(b)

        @pl.when(c >= NBUF)
        def _():
            store_wait(c - NBUF, b)

        extract(c, b)
        store_start(c, b)

        @pl.when(c + NBUF < NCHB)
        def _():
            make_pairs(c + NBUF, b)
            gather_start(b)

    @pl.loop(NCHB - NBUF, NCHB)
    def _tail(c):
        store_wait(c, c % NBUF)


def kernel(indices, table):
    idx3 = indices.reshape(NW, NCHB, CHUNK).astype(jnp.int32)
    tab2 = table.reshape(NUM_EMB // 2, PDIM)
    return _gather_kernel(idx3, tab2)


# speedup vs baseline: 1.9280x; 1.9280x over previous
import functools

import jax
import jax.numpy as jnp
from jax import lax
from jax.experimental import pallas as pl
from jax.experimental.pallas import tpu as pltpu
from jax.experimental.pallas import tpu_sc as plsc

NUM_EMB = 1000000
DIM = 64
B, F = 16384, 26
FLAT = B * F
NC, NS = 2, 16
NW = NC * NS
PER_W = FLAT // NW
SW = B // NW
SCH = 16
CHUNK = SCH * F
NCHUNK = SW // SCH
NBUF = 2

_mesh = plsc.VectorSubcoreMesh(core_axis_name="c", subcore_axis_name="s")


@functools.partial(
    pl.kernel,
    out_type=jax.ShapeDtypeStruct((B, F, DIM), jnp.float32),
    mesh=_mesh,
    scratch_types=[
        pltpu.VMEM((NCHUNK, CHUNK), jnp.int32),
        pltpu.VMEM((NBUF, CHUNK, DIM), jnp.float32),
        pltpu.SemaphoreType.DMA((NBUF,)),
        pltpu.SemaphoreType.DMA((NBUF,)),
    ],
    compiler_params=pltpu.CompilerParams(use_tc_tiling_on_sc=False),
)
def _gather_kernel(idx_hbm, table_hbm, out_hbm, idx_v, rows_v, gsem, ssem):
    wid = lax.axis_index("s") * NC + lax.axis_index("c")
    s_base = wid * SW

    pltpu.sync_copy(idx_hbm.at[wid], idx_v)

    def gather_start(chunk, buf):
        pltpu.async_copy(table_hbm.at[idx_v.at[chunk]],
                         rows_v.at[buf],
                         gsem.at[buf])

    def gather_wait(chunk, buf):
        pltpu.make_async_copy(table_hbm.at[idx_v.at[chunk]],
                              rows_v.at[buf],
                              gsem.at[buf]).wait()

    def store_start(chunk, buf):
        s0 = s_base + chunk * SCH
        for i in range(SCH):
            pltpu.async_copy(rows_v.at[buf, pl.ds(i * F, F)],
                             out_hbm.at[s0 + i], ssem.at[buf])

    def store_wait(chunk, buf):
        s0 = s_base + chunk * SCH
        for i in range(SCH):
            pltpu.make_async_copy(rows_v.at[buf, pl.ds(i * F, F)],
                                  out_hbm.at[s0 + i], ssem.at[buf]).wait()

    for b in range(NBUF):
        gather_start(b, b)

    @pl.loop(0, NCHUNK, step=NBUF)
    def _grp(g):
        for b in range(NBUF):
            chunk = g + b
            gather_wait(chunk, b)
            store_start(chunk, b)
            nxt = chunk + NBUF

            @pl.when(nxt < NCHUNK)
            def _():
                store_wait(chunk, b)
                gather_start(nxt, b)

    for b in range(NBUF):
        store_wait(NCHUNK - NBUF + b, b)


def kernel(indices, table):
    idx3 = indices.reshape(NW, NCHUNK, CHUNK).astype(jnp.int32)
    return _gather_kernel(idx3, table)
